# baseline (speedup 1.0000x reference)
import functools

import jax
import jax.numpy as jnp
from jax import lax
from jax.experimental import pallas as pl
from jax.experimental.pallas import tpu as pltpu
from jax.experimental.pallas import tpu_sc as plsc

NUM_FIELDS = 26
VOCAB = 100000
D_MODEL = 32
B, C, T = 1024, 1, 20
N = B * C * T * NUM_FIELDS
PACK = 128 // D_MODEL
NW = 32
PER_W = N // NW
CHUNK = 208
NCH = PER_W // CHUNK
NSLOT = 3
L = 16
SL = CHUNK // L
OCH = CHUNK // PACK

_mesh = plsc.VectorSubcoreMesh(core_axis_name="c", subcore_axis_name="s")


@functools.partial(
    pl.kernel,
    mesh=_mesh,
    out_type=jax.ShapeDtypeStruct((N * D_MODEL,), jnp.float32),
    scratch_types=[
        pltpu.VMEM((CHUNK,), jnp.int32),
        pltpu.VMEM((CHUNK,), jnp.int32),
        pltpu.VMEM((CHUNK,), jnp.int32),
        pltpu.VMEM((CHUNK,), jnp.int32),
        pltpu.VMEM((CHUNK,), jnp.int32),
        pltpu.VMEM((CHUNK,), jnp.int32),
        pltpu.VMEM((CHUNK, 128), jnp.float32),
        pltpu.VMEM((CHUNK, 128), jnp.float32),
        pltpu.VMEM((CHUNK, 128), jnp.float32),
        pltpu.VMEM((CHUNK * D_MODEL,), jnp.float32),
        pltpu.VMEM((CHUNK * D_MODEL,), jnp.float32),
        pltpu.VMEM((CHUNK * D_MODEL,), jnp.float32),
        pltpu.SemaphoreType.DMA,
        pltpu.SemaphoreType.DMA,
        pltpu.SemaphoreType.DMA,
        pltpu.SemaphoreType.DMA,
        pltpu.SemaphoreType.DMA,
        pltpu.SemaphoreType.DMA,
    ],
    compiler_params=pltpu.CompilerParams(needs_layout_passes=False),
)
def _embed(x_hbm, tab_hbm, out_hbm, g0, g1, g2, q0, q1, q2,
           r0, r1, r2, ob0, ob1, ob2, gs0, gs1, gs2, os0, os1, os2):
    g = (g0, g1, g2)
    q = (q0, q1, q2)
    rows = (r0, r1, r2)
    ob = (ob0, ob1, ob2)
    gsem = (gs0, gs1, gs2)
    osem = (os0, os1, os2)
    wid = lax.axis_index("s") * 2 + lax.axis_index("c")
    base = wid * PER_W
    obase = wid * PER_W * D_MODEL

    def stage_chunk(j, s):
        pltpu.sync_copy(x_hbm.at[pl.ds(base + j * CHUNK, CHUNK)], g[s])

        def tr(t, carry):
            span = pl.ds(t * L, L)
            xv = g[s][span]
            f = lax.rem(t * L + lax.iota(jnp.int32, L), NUM_FIELDS)
            q[s][span] = lax.bitwise_and(xv, PACK - 1)
            g[s][span] = f * (VOCAB // PACK) + lax.shift_right_logical(xv, 2)
            return carry

        lax.fori_loop(0, SL, tr, 0)
        pltpu.async_copy(tab_hbm.at[g[s]], rows[s], gsem[s])

    def wait_gather(s):
        pltpu.make_async_copy(tab_hbm.at[g[s]], rows[s], gsem[s]).wait()

    def start_out(j, s):
        pltpu.async_copy(
            ob[s],
            out_hbm.at[pl.ds(obase + j * CHUNK * D_MODEL, CHUNK * D_MODEL)],
            osem[s])

    def wait_out(j, s):
        pltpu.make_async_copy(
            ob[s],
            out_hbm.at[pl.ds(obase + j * CHUNK * D_MODEL, CHUNK * D_MODEL)],
            osem[s]
        ).wait()

    def select_chunk(s):
        def sel(t, carry):
            iv = t * L + lax.iota(jnp.int32, L)
            colb = q[s][pl.ds(t * L, L)] * D_MODEL
            ow = iv * D_MODEL
            for d in range(D_MODEL):
                val = plsc.load_gather(rows[s], [iv, colb + d])
                plsc.store_scatter(ob[s], [ow + d], val)
            return carry

        lax.fori_loop(0, SL, sel, 0)

    def step(k, s, do_stage, do_wait_out):
        if do_stage:
            stage_chunk(k + 2, (s + 2) % NSLOT)
        wait_gather(s)
        if do_wait_out:
            wait_out(k - NSLOT, s)
        select_chunk(s)
        start_out(k, s)

    stage_chunk(0, 0)
    stage_chunk(1, 1)
    step(0, 0, True, False)
    step(1, 1, True, False)
    step(2, 2, True, False)

    def superstep(i, carry):
        for c in range(NSLOT):
            step(3 + i * NSLOT + c, c, True, True)
        return carry

    lax.fori_loop(0, (NCH - 5) // NSLOT, superstep, 0)

    step(NCH - 2, (NCH - 2) % NSLOT, False, True)
    step(NCH - 1, (NCH - 1) % NSLOT, False, True)
    for j in range(NCH - NSLOT, NCH):
        wait_out(j, j % NSLOT)


def kernel(x, tables):
    xf = x.reshape(N).astype(jnp.int32)
    tf = tables.reshape(NUM_FIELDS * VOCAB // PACK, 128)
    out = _embed(xf, tf)
    return out.reshape(B, C, T, NUM_FIELDS, D_MODEL)


# speedup vs baseline: 1.4023x; 1.4023x over previous
import functools

import jax
import jax.numpy as jnp
from jax import lax
from jax.experimental import pallas as pl
from jax.experimental.pallas import tpu as pltpu
from jax.experimental.pallas import tpu_sc as plsc

NUM_FIELDS = 26
VOCAB = 100000
D_MODEL = 32
B, C, T = 1024, 1, 20
N = B * C * T * NUM_FIELDS
NBT = B * C * T
NW = 32
PER_W = NBT // NW
NBUF = 4
L = 16
SL = PER_W // L

_mesh = plsc.VectorSubcoreMesh(core_axis_name="c", subcore_axis_name="s")


@functools.partial(
    pl.kernel,
    mesh=_mesh,
    out_type=jax.ShapeDtypeStruct((N, D_MODEL), jnp.float32),
    scratch_types=[
        pltpu.VMEM((PER_W,), jnp.int32),
        pltpu.VMEM((PER_W,), jnp.int32),
        pltpu.VMEM((PER_W,), jnp.int32),
        pltpu.VMEM((PER_W,), jnp.int32),
        pltpu.VMEM((PER_W,), jnp.int32),
        pltpu.VMEM((PER_W,), jnp.int32),
        pltpu.VMEM((PER_W,), jnp.int32),
        pltpu.VMEM((PER_W,), jnp.int32),
        pltpu.VMEM((PER_W, D_MODEL), jnp.float32),
        pltpu.VMEM((PER_W, D_MODEL), jnp.float32),
        pltpu.VMEM((PER_W, D_MODEL), jnp.float32),
        pltpu.VMEM((PER_W, D_MODEL), jnp.float32),
        pltpu.SemaphoreType.DMA,
        pltpu.SemaphoreType.DMA,
        pltpu.SemaphoreType.DMA,
        pltpu.SemaphoreType.DMA,
        pltpu.SemaphoreType.DMA,
        pltpu.SemaphoreType.DMA,
        pltpu.SemaphoreType.DMA,
        pltpu.SemaphoreType.DMA,
    ],
    compiler_params=pltpu.CompilerParams(use_tc_tiling_on_sc=False),
)
def _embed(x_hbm, tab_hbm, out_hbm, i0, i1, i2, i3, o0, o1, o2, o3,
           r0, r1, r2, r3, g0, g1, g2, g3, s0, s1, s2, s3):
    idxb = (i0, i1, i2, i3)
    oidx = (o0, o1, o2, o3)
    rows = (r0, r1, r2, r3)
    gsem = (g0, g1, g2, g3)
    osem = (s0, s1, s2, s3)
    wid = lax.axis_index("s") * 2 + lax.axis_index("c")
    jbase = wid * PER_W

    def stage(f, b):
        pltpu.sync_copy(x_hbm.at[f].at[pl.ds(jbase, PER_W)], idxb[b])

        def mk(t, carry):
            jv = t * L + lax.iota(jnp.int32, L)
            oidx[b][pl.ds(t * L, L)] = (jbase + jv) * NUM_FIELDS + f
            return carry

        lax.fori_loop(0, SL, mk, 0)

    def start_gather(f, b):
        pltpu.async_copy(tab_hbm.at[f].at[idxb[b]], rows[b], gsem[b])

    def wait_gather(f, b):
        pltpu.make_async_copy(
            tab_hbm.at[f].at[idxb[b]], rows[b], gsem[b]).wait()

    def start_out(b):
        pltpu.async_copy(rows[b], out_hbm.at[oidx[b]], osem[b])

    def wait_out(b):
        pltpu.make_async_copy(rows[b], out_hbm.at[oidx[b]], osem[b]).wait()

    def step(f, b, first, last):
        if not last:
            bn = (b + 2) % NBUF
            if not first:
                wait_out(bn)
            stage(f + 2, bn)
            start_gather(f + 2, bn)
        wait_gather(f, b)
        start_out(b)

    stage(0, 0)
    stage(1, 1)
    start_gather(0, 0)
    start_gather(1, 1)
    for f in range(NUM_FIELDS):
        step(f, f % NBUF, f < 2, f >= NUM_FIELDS - 2)
    for f in range(NUM_FIELDS - NBUF, NUM_FIELDS):
        wait_out(f % NBUF)


def kernel(x, tables):
    xt = jnp.transpose(x.reshape(B * C * T, NUM_FIELDS)).astype(jnp.int32)
    out = _embed(xt, tables)
    return out.reshape(B, C, T, NUM_FIELDS, D_MODEL)
